# Initial kernel scaffold; baseline (speedup 1.0000x reference)
import jax
import jax.numpy as jnp
from jax.experimental import pallas as pl


def kernel(indices, weight):
    raise NotImplementedError("write your pallas kernel here")


# speedup vs baseline: 1.1027x; 1.1027x over previous
import functools

import jax
import jax.numpy as jnp
from jax import lax
from jax.experimental import pallas as pl
from jax.experimental.pallas import tpu as pltpu
from jax.experimental.pallas import tpu_sc as plsc

_DIM = 32


@functools.cache
def _make_gather(B, D, C):
    info = plsc.get_sparse_core_info()
    NC, NS = info.num_cores, info.num_subcores
    NW = NC * NS
    assert B % NW == 0
    b_per_w = B // NW
    assert b_per_w % C == 0
    n_chunks = b_per_w // C
    mesh = plsc.VectorSubcoreMesh(core_axis_name="c", subcore_axis_name="s")

    @functools.partial(
        pl.kernel,
        mesh=mesh,
        out_type=jax.ShapeDtypeStruct((B, D), jnp.float32),
        scratch_types=[
            pltpu.VMEM((C,), jnp.int32),
            pltpu.VMEM((C, D), jnp.float32),
            pltpu.SemaphoreType.DMA,
        ],
        compiler_params=pltpu.CompilerParams(use_tc_tiling_on_sc=False),
    )
    def k(idx_hbm, table_hbm, out_hbm, idx_v, rows_v, sem):
        wid = lax.axis_index("s") * NC + lax.axis_index("c")
        w_base = wid * b_per_w

        def body(i, carry):
            base = w_base + i * C
            pltpu.sync_copy(idx_hbm.at[pl.ds(base, C)], idx_v)
            pltpu.async_copy(table_hbm.at[idx_v], rows_v, sem).wait()
            pltpu.sync_copy(rows_v, out_hbm.at[pl.ds(base, C)])
            return carry

        lax.fori_loop(0, n_chunks, body, 0)

    return k


def kernel(indices, weight):
    S0, S1 = indices.shape
    B = S0 * S1
    idx_flat = indices.reshape(B).astype(jnp.int32)
    out = _make_gather(B, _DIM, 1600)(idx_flat, weight)
    return out.reshape(S0, S1, _DIM)


# speedup vs baseline: 1.1131x; 1.0094x over previous
import functools

import jax
import jax.numpy as jnp
from jax import lax
from jax.experimental import pallas as pl
from jax.experimental.pallas import tpu as pltpu
from jax.experimental.pallas import tpu_sc as plsc

_DIM = 32


@functools.cache
def _make_gather(B, D, C):
    info = plsc.get_sparse_core_info()
    NC, NS = info.num_cores, info.num_subcores
    NW = NC * NS
    assert B % NW == 0
    b_per_w = B // NW
    assert b_per_w % C == 0 and C % 8 == 0
    n = b_per_w // C
    mesh = plsc.VectorSubcoreMesh(core_axis_name="c", subcore_axis_name="s")

    @functools.partial(
        pl.kernel,
        mesh=mesh,
        out_type=jax.ShapeDtypeStruct((B, D), jnp.float32),
        scratch_types=[
            pltpu.VMEM((b_per_w,), jnp.int32),
            pltpu.VMEM((C, D), jnp.float32),
            pltpu.VMEM((C, D), jnp.float32),
            pltpu.SemaphoreType.DMA,
            pltpu.SemaphoreType.DMA,
            pltpu.SemaphoreType.DMA,
            pltpu.SemaphoreType.DMA,
        ],
        compiler_params=pltpu.CompilerParams(use_tc_tiling_on_sc=False),
    )
    def k(idx_hbm, table_hbm, out_hbm, idx_v, rows0, rows1, gs0, gs1, ws0, ws1):
        wid = lax.axis_index("s") * NC + lax.axis_index("c")
        w_base = wid * b_per_w
        pltpu.sync_copy(idx_hbm.at[pl.ds(w_base, b_per_w)], idx_v)

        rows = (rows0, rows1)
        gs = (gs0, gs1)
        ws = (ws0, ws1)

        def gather(i):
            return pltpu.async_copy(
                table_hbm.at[idx_v.at[pl.ds(i * C, C)]], rows[i % 2], gs[i % 2]
            )

        def put(i):
            return pltpu.async_copy(
                rows[i % 2], out_hbm.at[pl.ds(w_base + i * C, C)], ws[i % 2]
            )

        g = [None] * n
        w = [None] * n
        g[0] = gather(0)
        for i in range(n):
            if i + 1 < n:
                if i >= 1:
                    w[i - 1].wait()
                g[i + 1] = gather(i + 1)
            g[i].wait()
            w[i] = put(i)
        if n >= 2:
            w[n - 2].wait()
        w[n - 1].wait()

    return k


def kernel(indices, weight):
    S0, S1 = indices.shape
    B = S0 * S1
    idx_flat = indices.reshape(B).astype(jnp.int32)
    out = _make_gather(B, _DIM, 1280)(idx_flat, weight)
    return out.reshape(S0, S1, _DIM)


# speedup vs baseline: 1.8130x; 1.6288x over previous
import functools

import jax
import jax.numpy as jnp
from jax import lax
from jax.experimental import pallas as pl
from jax.experimental.pallas import tpu as pltpu
from jax.experimental.pallas import tpu_sc as plsc

_NBUF = 8
_LEAD = 5


@functools.cache
def _make_lookup(S0, S1, V, D, CB):
    info = plsc.get_sparse_core_info()
    NC, NS = info.num_cores, info.num_subcores
    NW = NC * NS
    assert S0 % NW == 0
    C = S0 // NW
    assert C % CB == 0
    H = C // CB
    n = S1 * H
    mesh = plsc.VectorSubcoreMesh(core_axis_name="c", subcore_axis_name="s")

    @functools.partial(
        pl.kernel,
        mesh=mesh,
        out_type=jax.ShapeDtypeStruct((S0, S1, D), jnp.float32),
        scratch_types=[
            pltpu.VMEM((S1, C), jnp.int32),
            *[pltpu.VMEM((CB, D), jnp.float32) for _ in range(_NBUF)],
            *[pltpu.SemaphoreType.DMA for _ in range(2 * _NBUF)],
        ],
        compiler_params=pltpu.CompilerParams(use_tc_tiling_on_sc=False),
    )
    def k(idx_hbm, table_hbm, out_hbm, idx_v, *bufs_and_sems):
        rows = bufs_and_sems[:_NBUF]
        gs = bufs_and_sems[_NBUF : 2 * _NBUF]
        ws = bufs_and_sems[2 * _NBUF :]
        wid = lax.axis_index("s") * NC + lax.axis_index("c")
        b0 = wid * C
        pltpu.sync_copy(idx_hbm.at[:, pl.ds(b0, C)], idx_v)

        def gather(t):
            s, h = t // H, t % H
            return pltpu.async_copy(
                table_hbm.at[idx_v.at[s, pl.ds(h * CB, CB)]],
                rows[t % _NBUF],
                gs[t % _NBUF],
            )

        def put(t):
            s, h = t // H, t % H
            return pltpu.async_copy(
                rows[t % _NBUF],
                out_hbm.at[pl.ds(b0 + h * CB, CB), s],
                ws[t % _NBUF],
            )

        g = {}
        w = {}
        waited = set()

        def drain(i):
            if i >= 0 and i in w and i not in waited:
                w[i].wait()
                waited.add(i)

        for t in range(min(_LEAD, n)):
            g[t] = gather(t)
        for s in range(n):
            g[s].wait()
            w[s] = put(s)
            t = s + _LEAD
            if t < n:
                drain(t - _NBUF)
                g[t] = gather(t)
        for s in range(n):
            drain(s)

    return k


def kernel(indices, weight):
    S0, S1 = indices.shape
    V, D = weight.shape
    return _make_lookup(S0, S1, V, D, 256)(indices.T, weight)


# speedup vs baseline: 2.3427x; 1.2922x over previous
import functools

import jax
import jax.numpy as jnp
from jax import lax
from jax.experimental import pallas as pl
from jax.experimental.pallas import tpu as pltpu
from jax.experimental.pallas import tpu_sc as plsc

_NBUF = 4
_CB = 128
_TP = 133
_J = 4


@functools.cache
def _make_lookup(S0, S1, V, D):
    info = plsc.get_sparse_core_info()
    NC, NS = info.num_cores, info.num_subcores
    NW = NC * NS
    L = info.num_lanes
    assert S0 % (NW * _CB) == 0 and D % 8 == 0 and _CB % L == 0
    C = S0 // NW
    H = C // _CB
    n = S1 * H
    DT = D // 8
    BT = S0 // 128
    assert n % _J == 0 and n >= 3 * _J
    mesh = plsc.VectorSubcoreMesh(core_axis_name="c", subcore_axis_name="s")

    @functools.partial(
        pl.kernel,
        mesh=mesh,
        out_type=jax.ShapeDtypeStruct((S1, DT, BT, 8, 128), jnp.float32),
        scratch_types=[
            pltpu.VMEM((S1, C), jnp.int32),
            *[pltpu.VMEM((_CB, D), jnp.float32) for _ in range(_NBUF)],
            *[pltpu.VMEM((DT, 8, _TP), jnp.float32) for _ in range(2)],
            *[pltpu.SemaphoreType.DMA for _ in range(_NBUF + 2)],
        ],
        compiler_params=pltpu.CompilerParams(
            use_tc_tiling_on_sc=False, needs_layout_passes=False
        ),
    )
    def k(idx_hbm, table_hbm, out_hbm, idx_v, *rest):
        rows = rest[:_NBUF]
        tb = rest[_NBUF : _NBUF + 2]
        gs = rest[_NBUF + 2 : 2 * _NBUF + 2]
        ws = rest[2 * _NBUF + 2 :]
        wid = lax.axis_index("s") * NC + lax.axis_index("c")
        b0 = wid * C
        bt0 = wid * H
        pltpu.sync_copy(idx_hbm.at[:, pl.ds(b0, C)], idx_v)

        lane = lax.iota(jnp.int32, L)
        halves = []
        for h in range(D // L):
            d_all = lane + h * L
            halves.append((d_all >> 3, d_all & 7))

        def gather(t, slot):
            s = t // H
            h = t - s * H
            return pltpu.async_copy(
                table_hbm.at[idx_v.at[s, pl.ds(h * _CB, _CB)]],
                rows[slot],
                gs[slot],
            )

        def wait_gather(slot):
            pltpu.make_async_copy(
                table_hbm.at[idx_v.at[0, pl.ds(0, _CB)]], rows[slot], gs[slot]
            ).wait()

        def put(t, slot):
            s = t // H
            h = t - s * H
            return pltpu.async_copy(
                tb[slot].at[:, :, pl.ds(0, 128)],
                out_hbm.at[s, :, bt0 + h],
                ws[slot],
            )

        def wait_put(slot):
            pltpu.make_async_copy(
                tb[slot].at[:, :, pl.ds(0, 128)], out_hbm.at[0, :, 0], ws[slot]
            ).wait()

        def transpose(rslot, tslot):
            src = rows[rslot]
            dst = tb[tslot]
            for b in range(_CB):
                for h in range(D // L):
                    v = src[b, pl.ds(h * L, L)]
                    i0, i1 = halves[h]
                    i2 = jnp.full((L,), b, jnp.int32)
                    plsc.store_scatter(dst, [i0, i1, i2], v)

        for t in range(_NBUF):
            gather(t, t)
        for t in range(_J):
            wait_gather(t % _NBUF)
            if t >= 2:
                wait_put(t % 2)
            transpose(t % _NBUF, t % 2)
            put(t, t % 2)
            gather(t + _NBUF, t % _NBUF)

        def body(i, carry):
            t0 = i * _J
            for j in range(_J):
                t = t0 + j
                slot = j % _NBUF
                ts = j % 2
                wait_gather(slot)
                wait_put(ts)
                transpose(slot, ts)
                put(t, ts)
                nt = jnp.minimum(t + _NBUF, n - 1)
                gather(nt, slot)
            return carry

        lax.fori_loop(1, n // _J - 1, body, 0)

        for t in range(n - _J, n):
            wait_gather(t % _NBUF)
            wait_put(t % 2)
            transpose(t % _NBUF, t % 2)
            put(t, t % 2)
        wait_put(0)
        wait_put(1)

    return k


def kernel(indices, weight):
    S0, S1 = indices.shape
    V, D = weight.shape
    r = _make_lookup(S0, S1, V, D)(indices.T, weight)
    return r.transpose(2, 4, 0, 1, 3).reshape(S0, S1, D)
